# baseline (speedup 1.0000x reference)
import functools

import jax
import jax.numpy as jnp
from jax import lax
from jax.experimental import pallas as pl
from jax.experimental.pallas import tpu as pltpu
from jax.experimental.pallas import tpu_sc as plsc

_VOCAB = 1000
_D = 128
_B = 1024
_L = 50
_LP = 64
_CB = 64
_NBC = _B // _CB
_NCHT = _LP * _NBC
_NW = 32
_NCH = _NCHT // _NW


def _make_gather():
    mesh = plsc.VectorSubcoreMesh(core_axis_name="c", subcore_axis_name="s")

    @functools.partial(
        pl.kernel,
        out_type=jax.ShapeDtypeStruct((_NCHT, _CB, _D), jnp.float32),
        mesh=mesh,
        scratch_types=[
            pltpu.VMEM((_NCH, _CB), jnp.int32),
            pltpu.VMEM((_CB, _D), jnp.float32),
            pltpu.VMEM((_CB, _D), jnp.float32),
            pltpu.SemaphoreType.DMA,
            pltpu.SemaphoreType.DMA,
        ],
    )
    def gather(ids_hbm, emb_hbm, x_hbm, idx_v, buf0, buf1, sem0, sem1):
        wid = lax.axis_index("s") * 2 + lax.axis_index("c")
        base = wid * _NCH
        pltpu.sync_copy(ids_hbm.at[pl.ds(base, _NCH)], idx_v)

        def g(c, buf, sem):
            return pltpu.make_async_copy(emb_hbm.at[idx_v.at[c]], buf, sem)

        def w(c, buf):
            pltpu.sync_copy(buf, x_hbm.at[base + c])

        g(0, buf0, sem0).start()

        def body(p, carry):
            a = 2 * p
            g(a + 1, buf1, sem1).start()
            g(a, buf0, sem0).wait()
            w(a, buf0)
            g(a + 2, buf0, sem0).start()
            g(a + 1, buf1, sem1).wait()
            w(a + 1, buf1)
            return carry

        lax.fori_loop(0, _NCH // 2 - 1, body, 0)
        g(_NCH - 1, buf1, sem1).start()
        g(_NCH - 2, buf0, sem0).wait()
        w(_NCH - 2, buf0)
        g(_NCH - 1, buf1, sem1).wait()
        w(_NCH - 1, buf1)

    return gather


_gather = _make_gather()


def _head_body(x_ref, w_ref, b_ref, out_ref):
    xl = x_ref[...].reshape(_B, _D)
    res = lax.dot_general(
        w_ref[...], xl, (((1,), (1,)), ((), ())),
        preferred_element_type=jnp.float32)
    out_ref[...] = (res + b_ref[...]).reshape(1, _VOCAB, _B)


def _head(x, w, b2d):
    return pl.pallas_call(
        _head_body,
        grid=(_L,),
        in_specs=[
            pl.BlockSpec((_NBC, _CB, _D), lambda i: (i, 0, 0)),
            pl.BlockSpec((_VOCAB, _D), lambda i: (0, 0)),
            pl.BlockSpec((_VOCAB, 1), lambda i: (0, 0)),
        ],
        out_specs=pl.BlockSpec((1, _VOCAB, _B), lambda i: (i, 0, 0)),
        out_shape=jax.ShapeDtypeStruct((_L, _VOCAB, _B), jnp.float32),
    )(x, w, b2d)


def kernel(input_ids, emb_table, head_w, head_b):
    ids = input_ids.astype(jnp.int32)
    ids = jnp.pad(ids, ((0, 0), (0, _LP - _L)))
    ids_t = ids.T.reshape(_NCHT, _CB)
    x = _gather(ids_t, emb_table)
    out_t = _head(x, head_w, head_b.reshape(_VOCAB, 1))
    return jnp.transpose(out_t, (2, 0, 1))


# speedup vs baseline: 5.1190x; 5.1190x over previous
import functools

import jax
import jax.numpy as jnp
from jax import lax
from jax.experimental import pallas as pl
from jax.experimental.pallas import tpu as pltpu
from jax.experimental.pallas import tpu_sc as plsc

_VOCAB = 1000
_D = 128
_B = 1024
_L = 50
_CB = 64
_NBC = _B // _CB
_NCHT = _L * _NBC
_NW = 32
_NCH = _NCHT // _NW


def _make_gather():
    mesh = plsc.VectorSubcoreMesh(core_axis_name="c", subcore_axis_name="s")

    @functools.partial(
        pl.kernel,
        out_type=jax.ShapeDtypeStruct((_NCHT, _CB, _D), jnp.float32),
        mesh=mesh,
        scratch_types=[
            pltpu.VMEM((_NCH, _CB), jnp.int32),
            pltpu.VMEM((_CB, _D), jnp.float32),
            pltpu.VMEM((_CB, _D), jnp.float32),
            pltpu.SemaphoreType.DMA,
            pltpu.SemaphoreType.DMA,
        ],
    )
    def gather(ids_hbm, emb_hbm, x_hbm, idx_v, buf0, buf1, sem0, sem1):
        wid = lax.axis_index("s") * 2 + lax.axis_index("c")
        base = wid * _NCH
        pltpu.sync_copy(ids_hbm.at[wid], idx_v)

        def g(c, buf, sem):
            return pltpu.make_async_copy(emb_hbm.at[idx_v.at[c]], buf, sem)

        def w(c, buf):
            pltpu.sync_copy(buf, x_hbm.at[base + c])

        g(0, buf0, sem0).start()

        def body(p, carry):
            a = 2 * p
            g(a + 1, buf1, sem1).start()
            g(a, buf0, sem0).wait()
            w(a, buf0)
            g(a + 2, buf0, sem0).start()
            g(a + 1, buf1, sem1).wait()
            w(a + 1, buf1)
            return carry

        lax.fori_loop(0, (_NCH - 3) // 2, body, 0)
        g(_NCH - 2, buf1, sem1).start()
        g(_NCH - 3, buf0, sem0).wait()
        w(_NCH - 3, buf0)
        g(_NCH - 1, buf0, sem0).start()
        g(_NCH - 2, buf1, sem1).wait()
        w(_NCH - 2, buf1)
        g(_NCH - 1, buf0, sem0).wait()
        w(_NCH - 1, buf0)

    return gather


_gather = _make_gather()


def _head_body(x_ref, w_ref, b_ref, out_ref):
    xl = x_ref[...].reshape(_B, _D)
    res = lax.dot_general(
        w_ref[...], xl, (((1,), (1,)), ((), ())),
        preferred_element_type=jnp.float32)
    out_ref[...] = (res + b_ref[...]).reshape(1, _VOCAB, _B)


def _head(x, w, b2d):
    return pl.pallas_call(
        _head_body,
        grid=(_L,),
        in_specs=[
            pl.BlockSpec((_NBC, _CB, _D), lambda i: (i, 0, 0)),
            pl.BlockSpec((_VOCAB, _D), lambda i: (0, 0)),
            pl.BlockSpec((_VOCAB, 1), lambda i: (0, 0)),
        ],
        out_specs=pl.BlockSpec((1, _VOCAB, _B), lambda i: (i, 0, 0)),
        out_shape=jax.ShapeDtypeStruct((_L, _VOCAB, _B), jnp.float32),
    )(x, w, b2d)


def kernel(input_ids, emb_table, head_w, head_b):
    ids = input_ids.astype(jnp.int32)
    ids_t = ids.T.reshape(_NW, _NCH, _CB)
    x = _gather(ids_t, emb_table)
    out_t = _head(x, head_w, head_b.reshape(_VOCAB, 1))
    return jnp.transpose(out_t, (2, 0, 1))
